# baseline (speedup 1.0000x reference)
import functools

import jax
import jax.numpy as jnp
from jax import lax
from jax.experimental import pallas as pl
from jax.experimental.pallas import tpu as pltpu
from jax.experimental.pallas import tpu_sc as plsc

_LANES = 16
_NUM_WORKERS = 32
_CHUNK = 128


def _build_sc_call(n_rows, seq_len, d_model):
    rows_per_worker = n_rows // _NUM_WORKERS
    n_chunks = rows_per_worker // _CHUNK
    mesh = plsc.VectorSubcoreMesh(
        core_axis_name="c", subcore_axis_name="s", num_cores=2, num_subcores=16
    )

    @functools.partial(
        pl.kernel,
        out_type=jax.ShapeDtypeStruct((n_rows, d_model), jnp.float32),
        mesh=mesh,
        scratch_types=[
            pltpu.VMEM((rows_per_worker,), jnp.int32),
            pltpu.VMEM((2 * seq_len, d_model), jnp.float32),
            pltpu.VMEM((_CHUNK, d_model), jnp.float32),
            pltpu.SemaphoreType.DMA,
        ],
        compiler_params=pltpu.CompilerParams(use_tc_tiling_on_sc=False),
    )
    def sc_encode(idx_hbm, table_hbm, pos2_hbm, out_hbm, idx_v, pos_v, buf, sem):
        wid = lax.axis_index("s") * 2 + lax.axis_index("c")
        base = pl.multiple_of(wid * rows_per_worker, rows_per_worker)
        pltpu.sync_copy(idx_hbm.at[pl.ds(base, rows_per_worker)], idx_v)
        pltpu.sync_copy(pos2_hbm, pos_v)

        def chunk_body(c, carry):
            rowstart = pl.multiple_of(c * _CHUNK, _CHUNK)
            pltpu.async_copy(
                table_hbm.at[idx_v.at[pl.ds(rowstart, _CHUNK)]], buf, sem
            ).wait()
            p0 = lax.rem(rowstart, seq_len)

            def add_body(r, carry2):
                for k in range(d_model // _LANES):
                    sl = pl.ds(k * _LANES, _LANES)
                    buf[r, sl] = buf[r, sl] + pos_v[p0 + r, sl]
                return carry2

            lax.fori_loop(0, _CHUNK, add_body, 0)
            pltpu.sync_copy(buf, out_hbm.at[pl.ds(base + rowstart, _CHUNK)])
            return carry

        lax.fori_loop(0, n_chunks, chunk_body, 0)

    return sc_encode


def kernel(inputs, emb_table, pos_table):
    batch, seq_len = inputs.shape
    d_model = emb_table.shape[1]
    n_rows = batch * seq_len
    idx_flat = inputs.reshape(n_rows)
    pos2 = jnp.concatenate([pos_table, pos_table], axis=0)
    out = _build_sc_call(n_rows, seq_len, d_model)(idx_flat, emb_table, pos2)
    return out.reshape(batch, seq_len, d_model)


# speedup vs baseline: 1.2263x; 1.2263x over previous
import functools

import jax
import jax.numpy as jnp
from jax import lax
from jax.experimental import pallas as pl
from jax.experimental.pallas import tpu as pltpu
from jax.experimental.pallas import tpu_sc as plsc

_LANES = 16
_NUM_WORKERS = 32
_CHUNK = 128


def _build_sc_call(n_rows, seq_len, d_model):
    rpw = n_rows // _NUM_WORKERS
    n_chunks = rpw // _CHUNK
    n_vec = d_model // _LANES
    mesh = plsc.VectorSubcoreMesh(
        core_axis_name="c", subcore_axis_name="s", num_cores=2, num_subcores=16
    )

    @functools.partial(
        pl.kernel,
        out_type=jax.ShapeDtypeStruct((n_rows, d_model), jnp.float32),
        mesh=mesh,
        scratch_types=[
            pltpu.VMEM((rpw,), jnp.int32),
            pltpu.VMEM((2 * seq_len, d_model), jnp.float32),
            pltpu.VMEM((2, _CHUNK, 2 * d_model), jnp.float32),
            pltpu.VMEM((2, _CHUNK, d_model), jnp.float32),
            pltpu.SemaphoreType.DMA,
            pltpu.SemaphoreType.DMA,
        ],
        compiler_params=pltpu.CompilerParams(needs_layout_passes=False),
    )
    def sc_encode(idx_hbm, tab_hbm, pos2_hbm, out_hbm,
                  idx_v, pos_v, bufs, outbufs, sem0, sem1):
        sems = (sem0, sem1)
        wid = lax.axis_index("s") * 2 + lax.axis_index("c")
        base = pl.multiple_of(wid * rpw, rpw)
        pltpu.sync_copy(idx_hbm.at[pl.ds(base, rpw)], idx_v)
        pltpu.sync_copy(pos2_hbm, pos_v)

        def gather_start(c, b):
            off = pl.multiple_of(c * _CHUNK, _CHUNK)
            return pltpu.async_copy(
                tab_hbm.at[idx_v.at[pl.ds(off, _CHUNK)]], bufs.at[b], sems[b]
            )

        def gather_wait(c, b):
            off = pl.multiple_of(c * _CHUNK, _CHUNK)
            pltpu.make_async_copy(
                tab_hbm.at[idx_v.at[pl.ds(off, _CHUNK)]], bufs.at[b], sems[b]
            ).wait()

        def process(c, b):
            p0 = lax.rem(c * _CHUNK, seq_len)
            buf = bufs.at[b]
            outb = outbufs.at[b]

            def row_body(r, carry):
                pr = p0 + r
                for k in range(n_vec):
                    sl = pl.ds(k * _LANES, _LANES)
                    outb[r, sl] = buf[r, sl] + pos_v[pr, sl]
                return carry

            lax.fori_loop(0, _CHUNK, row_body, 0)
            pltpu.sync_copy(outb, out_hbm.at[pl.ds(base + c * _CHUNK, _CHUNK)])

        gather_start(0, 0)

        def chunk_pair(j, carry):
            for b in range(2):
                c = 2 * j + b
                gather_start(lax.rem(c + 1, n_chunks), 1 - b)
                gather_wait(c, b)
                process(c, b)
            return carry

        lax.fori_loop(0, n_chunks // 2, chunk_pair, 0)
        gather_wait(0, 0)

    return sc_encode


def kernel(inputs, emb_table, pos_table):
    batch, seq_len = inputs.shape
    d_model = emb_table.shape[1]
    n_rows = batch * seq_len
    idx_flat = inputs.reshape(n_rows)
    tab_pad = jnp.pad(emb_table, ((0, 0), (0, d_model)))
    pos2 = jnp.concatenate([pos_table, pos_table], axis=0)
    out = _build_sc_call(n_rows, seq_len, d_model)(idx_flat, tab_pad, pos2)
    return out.reshape(batch, seq_len, d_model)
